# Initial kernel scaffold; baseline (speedup 1.0000x reference)
import jax
import jax.numpy as jnp
from jax.experimental import pallas as pl


def kernel(position_ids, table):
    raise NotImplementedError("write your pallas kernel here")


# speedup vs baseline: 2.4892x; 2.4892x over previous
import functools

import jax
import jax.numpy as jnp
from jax import lax
from jax.experimental import pallas as pl
from jax.experimental.pallas import tpu as pltpu
from jax.experimental.pallas import tpu_sc as plsc

_NUM_CORES = 2
_NUM_SUBCORES = 16
_NW = _NUM_CORES * _NUM_SUBCORES

_CHUNK = 2048


@functools.cache
def _build(b_tot: int, vocab: int, d: int):
    assert b_tot % (_NW * _CHUNK) == 0
    b_per_w = b_tot // _NW
    n_chunks = b_per_w // _CHUNK
    mesh = plsc.VectorSubcoreMesh(core_axis_name="c", subcore_axis_name="s")

    @functools.partial(
        pl.kernel,
        out_type=jax.ShapeDtypeStruct((b_tot, d), jnp.float32),
        mesh=mesh,
        scratch_types=[
            pltpu.VMEM((_CHUNK,), jnp.int32),
            pltpu.VMEM((_CHUNK, d), jnp.float32),
            pltpu.SemaphoreType.DMA,
        ],
        compiler_params=pltpu.CompilerParams(use_tc_tiling_on_sc=False),
    )
    def gather_kernel(ids_hbm, table_hbm, out_hbm, idx_v, rows_v, sem):
        wid = lax.axis_index("s") * _NUM_CORES + lax.axis_index("c")
        base = wid * b_per_w

        def step(i, carry):
            off = pl.multiple_of(base + i * _CHUNK, _CHUNK)
            pltpu.sync_copy(ids_hbm.at[pl.ds(off, _CHUNK)], idx_v)
            pltpu.async_copy(table_hbm.at[idx_v], rows_v, sem).wait()
            pltpu.sync_copy(rows_v, out_hbm.at[pl.ds(off, _CHUNK)])
            return carry

        lax.fori_loop(0, n_chunks, step, 0)

    return gather_kernel


def kernel(position_ids, table):
    b, s = position_ids.shape
    vocab, d = table.shape
    ids_flat = position_ids.reshape(-1).astype(jnp.int32)
    out = _build(b * s, vocab, d)(ids_flat, table)
    return out.reshape(b, s, d)


# speedup vs baseline: 2.5710x; 1.0329x over previous
import functools

import jax
import jax.numpy as jnp
from jax import lax
from jax.experimental import pallas as pl
from jax.experimental.pallas import tpu as pltpu
from jax.experimental.pallas import tpu_sc as plsc

_NUM_CORES = 2
_NUM_SUBCORES = 16
_NW = _NUM_CORES * _NUM_SUBCORES

_CHUNK = 2048
_NBUF = 3


@functools.cache
def _build(b_tot: int, vocab: int, d: int):
    assert b_tot % (_NW * _CHUNK) == 0
    b_per_w = b_tot // _NW
    n = b_per_w // _CHUNK
    assert n >= 4
    mesh = plsc.VectorSubcoreMesh(core_axis_name="c", subcore_axis_name="s")

    @functools.partial(
        pl.kernel,
        out_type=jax.ShapeDtypeStruct((b_tot, d), jnp.float32),
        mesh=mesh,
        scratch_types=[
            pltpu.VMEM((_NBUF, _CHUNK), jnp.int32),
            pltpu.VMEM((_NBUF, _CHUNK, d), jnp.float32),
            pltpu.SemaphoreType.DMA((_NBUF,)),
            pltpu.SemaphoreType.DMA((_NBUF,)),
            pltpu.SemaphoreType.DMA((_NBUF,)),
        ],
        compiler_params=pltpu.CompilerParams(use_tc_tiling_on_sc=False),
    )
    def gather_kernel(ids_hbm, table_hbm, out_hbm, idx_v, rows_v, s_idx, s_gat, s_out):
        wid = lax.axis_index("s") * _NUM_CORES + lax.axis_index("c")
        base = wid * b_per_w

        def off(i):
            return pl.multiple_of(base + i * _CHUNK, _CHUNK)

        def fire_idx(i, b):
            pltpu.async_copy(ids_hbm.at[pl.ds(off(i), _CHUNK)], idx_v.at[b], s_idx.at[b])

        def wait_idx(i, b):
            pltpu.make_async_copy(
                ids_hbm.at[pl.ds(off(i), _CHUNK)], idx_v.at[b], s_idx.at[b]
            ).wait()

        def fire_gat(b):
            pltpu.async_copy(table_hbm.at[idx_v.at[b]], rows_v.at[b], s_gat.at[b])

        def wait_gat(b):
            pltpu.make_async_copy(
                table_hbm.at[idx_v.at[b]], rows_v.at[b], s_gat.at[b]
            ).wait()

        def fire_out(i, b):
            pltpu.async_copy(rows_v.at[b], out_hbm.at[pl.ds(off(i), _CHUNK)], s_out.at[b])

        def wait_out(i, b):
            pltpu.make_async_copy(
                rows_v.at[b], out_hbm.at[pl.ds(off(i), _CHUNK)], s_out.at[b]
            ).wait()

        fire_idx(0, 0)
        wait_idx(0, 0)
        fire_gat(0)
        fire_idx(1, 1)
        for i in (1, 2):
            b, bp = i % _NBUF, (i - 1) % _NBUF
            wait_idx(i, b)
            fire_gat(b)
            wait_gat(bp)
            fire_out(i - 1, bp)
            fire_idx(i + 1, (i + 1) % _NBUF)

        def step(i, carry):
            b = lax.rem(i, _NBUF)
            bp = lax.rem(i - 1, _NBUF)
            wait_idx(i, b)
            wait_out(i - _NBUF, b)
            fire_gat(b)
            wait_gat(bp)
            fire_out(i - 1, bp)
            fire_idx(i + 1, lax.rem(i + 1, _NBUF))
            return carry

        lax.fori_loop(3, n - 1, step, 0)

        i = n - 1
        b, bp = i % _NBUF, (i - 1) % _NBUF
        wait_idx(i, b)
        wait_out(i - _NBUF, b)
        fire_gat(b)
        wait_gat(bp)
        fire_out(i - 1, bp)
        wait_gat(b)
        fire_out(i, b)
        wait_out(n - 2, bp)
        wait_out(n - 1, b)
        wait_out(n - 3, (n - 3) % _NBUF)

    return gather_kernel


def kernel(position_ids, table):
    b, s = position_ids.shape
    vocab, d = table.shape
    ids_flat = position_ids.reshape(-1).astype(jnp.int32)
    out = _build(b * s, vocab, d)(ids_flat, table)
    return out.reshape(b, s, d)
